# baseline (speedup 1.0000x reference)
import functools

import jax
import jax.numpy as jnp
from jax import lax
from jax.experimental import pallas as pl
from jax.experimental.pallas import tpu as pltpu
from jax.experimental.pallas import tpu_sc as plsc

N = 100000
E = 1600000
IN_F = 8
HID = 32
HALF = 16
NCORE = 2
NSUB = 16

NP = 100096
NV = NP
ZROWS = NV // NSUB
WROWS = NV // NSUB

EP = 1638400
PAD = EP - E
ER = EP // 128

import numpy as _np
_PAD_SRC = ((_np.arange(PAD) * 2621) % N).astype(_np.int32)
_PAD_DST = (N + (_np.arange(PAD) % 16)).astype(_np.int32)

MS_SLAB = 4
MS_RPS = ER // NSUB
MS_NSLAB = MS_RPS // MS_SLAB

DG_SLAB = 2
DG_ROWS = E // 128
DG_RPW = 392
DG_NSLAB = DG_RPW // DG_SLAB

BN = 2944
GRID = NP // BN
BP = BN // 4
NPK = NP // 4
NP8 = NP // 8


_mesh = plsc.VectorSubcoreMesh(core_axis_name="c", subcore_axis_name="s")


def _deg_body(ei3, zeros, ones, degh, didx, ones_v, degS, sem_i, sem_s):
  c = lax.axis_index("c")
  s = lax.axis_index("s")
  pltpu.sync_copy(zeros, degS.at[pl.ds(s * ZROWS, ZROWS)])
  pltpu.sync_copy(ones, ones_v)
  row0 = (c * NSUB + s) * DG_RPW

  def valid(slab):
    return row0 + slab * DG_SLAB < DG_ROWS

  def idx_start(slab, q):
    pltpu.async_copy(
        ei3.at[1, pl.ds(row0 + slab * DG_SLAB, DG_SLAB)], didx.at[q],
        sem_i.at[q])

  def idx_wait(slab, q):
    pltpu.make_async_copy(
        ei3.at[1, pl.ds(row0 + slab * DG_SLAB, DG_SLAB)], didx.at[q],
        sem_i.at[q]).wait()

  def fire_scatters(b, q):
    for j in range(DG_SLAB):
      pltpu.async_copy(ones_v, degS.at[didx.at[q, j]], sem_s.at[b],
                       add=True)

  def drain_scatters(b, q):
    for j in range(DG_SLAB):
      pltpu.make_async_copy(ones_v, degS.at[didx.at[q, j]],
                            sem_s.at[b]).wait()

  @pl.when(valid(0))
  def _():
    idx_start(0, 0)

  @pl.when(valid(1))
  def _():
    idx_start(1, 1)

  plsc.subcore_barrier()

  def outer(g, carry):
    for k in range(4):
      i = g * 4 + k
      b = k % 2
      q = k % 4

      @pl.when(valid(i))
      def _():
        idx_wait(i, q)

      if k < 2:
        @pl.when(jnp.logical_and(g > 0, valid(i - 2)))
        def _():
          drain_scatters(b, (k - 2) % 4)
      else:
        @pl.when(valid(i - 2))
        def _():
          drain_scatters(b, (k - 2) % 4)

      @pl.when(valid(i))
      def _():
        fire_scatters(b, q)

      @pl.when(jnp.logical_and(i + 2 < DG_NSLAB, valid(i + 2)))
      def _():
        idx_start(i + 2, (k + 2) % 4)
    return carry

  lax.fori_loop(0, DG_NSLAB // 4, outer, 0)

  @pl.when(valid(DG_NSLAB - 2))
  def _():
    drain_scatters(0, 2)

  @pl.when(valid(DG_NSLAB - 1))
  def _():
    drain_scatters(1, 3)

  plsc.subcore_barrier()
  pltpu.sync_copy(degS.at[pl.ds(s * WROWS, WROWS)],
                  degh.at[c, pl.ds(s * WROWS, WROWS)])


_deg_call = pl.kernel(
    _deg_body,
    out_type=jax.ShapeDtypeStruct((NCORE, NP, HALF), jnp.float32),
    mesh=_mesh,
    compiler_params=pltpu.CompilerParams(use_tc_tiling_on_sc=False),
    scratch_types=[
        pltpu.VMEM((4, DG_SLAB, 128), jnp.int32),
        pltpu.VMEM((128, HALF), jnp.float32),
        pltpu.VMEM_SHARED((NV, HALF), jnp.float32),
        pltpu.SemaphoreType.DMA((4,)),
        pltpu.SemaphoreType.DMA((2,)),
    ],
)


def _msg_body(ycat, srcp1, dstp, zeros, acc, sidx, didx, rows, accS,
              sem_g, sem_i, sem_s):
  c = lax.axis_index("c")
  s = lax.axis_index("s")
  pltpu.sync_copy(zeros, accS.at[pl.ds(s * ZROWS, ZROWS)])
  row0 = s * MS_RPS

  off = c * NP

  def idx_start(slab, q):
    r0 = row0 + slab * MS_SLAB
    pltpu.async_copy(srcp1.at[pl.ds(r0, MS_SLAB)], sidx.at[q],
                     sem_i.at[q])
    pltpu.async_copy(dstp.at[pl.ds(r0, MS_SLAB)], didx.at[q], sem_i.at[q])

  def idx_wait(slab, q):
    r0 = row0 + slab * MS_SLAB
    pltpu.make_async_copy(srcp1.at[pl.ds(r0, MS_SLAB)], sidx.at[q],
                          sem_i.at[q]).wait()
    pltpu.make_async_copy(dstp.at[pl.ds(r0, MS_SLAB)], didx.at[q],
                          sem_i.at[q]).wait()
    for r in range(MS_SLAB):
      for t in range(8):
        sl = pl.ds(t * 16, 16)
        sidx[q, r, sl] = sidx[q, r, sl] + off

  def fire_gathers(b, q):
    for j in range(MS_SLAB):
      pltpu.async_copy(ycat.at[sidx.at[q, j]],
                       rows.at[b, pl.ds(j * 128, 128)], sem_g.at[b])

  def drain_gathers(b, q):
    for j in range(MS_SLAB):
      pltpu.make_async_copy(ycat.at[sidx.at[q, j]],
                            rows.at[b, pl.ds(j * 128, 128)],
                            sem_g.at[b]).wait()

  def fire_scatters(b, q):
    for j in range(MS_SLAB):
      pltpu.async_copy(rows.at[b, pl.ds(j * 128, 128)],
                       accS.at[didx.at[q, j]], sem_s.at[b], add=True)

  def drain_scatters(b, q):
    for j in range(MS_SLAB):
      pltpu.make_async_copy(rows.at[b, pl.ds(j * 128, 128)],
                            accS.at[didx.at[q, j]], sem_s.at[b]).wait()

  idx_start(0, 0)
  idx_wait(0, 0)
  fire_gathers(0, 0)
  idx_start(1, 1)
  plsc.subcore_barrier()

  def outer(g, carry):
    for k in range(4):
      i = g * 4 + k
      b = k % 2
      q = k % 4
      drain_gathers(b, q)
      if k == 0:
        @pl.when(g > 0)
        def _():
          drain_scatters(1 - b, (k - 1) % 4)
      else:
        drain_scatters(1 - b, (k - 1) % 4)
      if k == 3:
        @pl.when(g < MS_NSLAB // 4 - 1)
        def _():
          idx_wait(i + 1, (k + 1) % 4)
          fire_gathers(1 - b, (k + 1) % 4)
      else:
        idx_wait(i + 1, (k + 1) % 4)
        fire_gathers(1 - b, (k + 1) % 4)
      fire_scatters(b, q)

      @pl.when(i + 2 < MS_NSLAB)
      def _():
        idx_start(i + 2, (k + 2) % 4)
    return carry

  lax.fori_loop(0, MS_NSLAB // 4, outer, 0)
  drain_scatters(1, 3)
  plsc.subcore_barrier()
  pltpu.sync_copy(accS.at[pl.ds(s * WROWS, WROWS)],
                  acc.at[c, pl.ds(s * WROWS, WROWS)])


_msg_call = pl.kernel(
    _msg_body,
    out_type=jax.ShapeDtypeStruct((NCORE, NP, HALF), jnp.float32),
    mesh=_mesh,
    compiler_params=pltpu.CompilerParams(use_tc_tiling_on_sc=False),
    scratch_types=[
        pltpu.VMEM((4, MS_SLAB, 128), jnp.int32),
        pltpu.VMEM((4, MS_SLAB, 128), jnp.int32),
        pltpu.VMEM((2, MS_SLAB * 128, HALF), jnp.float32),
        pltpu.VMEM_SHARED((NV, HALF), jnp.float32),
        pltpu.SemaphoreType.DMA((2,)),
        pltpu.SemaphoreType.DMA((4,)),
        pltpu.SemaphoreType.DMA((2,)),
    ],
)


def _dense1_body(x8, degh4, w1bd, ypk, selfpk, dishpk):
  dis = lax.rsqrt(degh4[0] + degh4[1] + 1.0)
  y0 = jnp.dot(x8[...], w1bd[0], preferred_element_type=jnp.float32) * dis
  y1 = jnp.dot(x8[...], w1bd[1], preferred_element_type=jnp.float32) * dis
  ypk[...] = jnp.concatenate([y0[None], y1[None]], axis=0)
  selfpk[...] = jnp.concatenate([(y0 * dis)[None], (y1 * dis)[None]], axis=0)
  dishpk[...] = dis


BR = NP8 // GRID


def _tc_dense1(x8, degh4, w1bd):
  return pl.pallas_call(
      _dense1_body,
      grid=(GRID,),
      in_specs=[
          pl.BlockSpec((BR, 64), lambda i: (i, 0)),
          pl.BlockSpec((NCORE, BR, 128), lambda i: (0, i, 0)),
          pl.BlockSpec((NCORE, 64, 128), lambda i: (0, 0, 0)),
      ],
      out_specs=[
          pl.BlockSpec((NCORE, BR, 128), lambda i: (0, i, 0)),
          pl.BlockSpec((NCORE, BR, 128), lambda i: (0, i, 0)),
          pl.BlockSpec((BR, 128), lambda i: (i, 0)),
      ],
      out_shape=[
          jax.ShapeDtypeStruct((NCORE, NP8, 128), jnp.float32),
          jax.ShapeDtypeStruct((NCORE, NP8, 128), jnp.float32),
          jax.ShapeDtypeStruct((NP8, 128), jnp.float32),
      ],
  )(x8, degh4, w1bd)


def _dense2_body(acc, selfpk, dishpk, kbd, b1t, ypk2, selfpk2):
  dis = dishpk[...]
  h0 = jnp.maximum(dis * acc[0] + selfpk[0] + b1t[0:1], 0.0)
  h1 = jnp.maximum(dis * acc[1] + selfpk[1] + b1t[1:2], 0.0)
  y20 = (jnp.dot(h0, kbd[0, 0], preferred_element_type=jnp.float32)
         + jnp.dot(h1, kbd[1, 0], preferred_element_type=jnp.float32)) * dis
  y21 = (jnp.dot(h0, kbd[0, 1], preferred_element_type=jnp.float32)
         + jnp.dot(h1, kbd[1, 1], preferred_element_type=jnp.float32)) * dis
  ypk2[...] = jnp.concatenate([y20[None], y21[None]], axis=0)
  selfpk2[...] = jnp.concatenate([(y20 * dis)[None], (y21 * dis)[None]],
                                 axis=0)


def _tc_dense2(acc, selfpk, dishpk, kbd, b1t):
  return pl.pallas_call(
      _dense2_body,
      grid=(GRID,),
      in_specs=[
          pl.BlockSpec((NCORE, BR, 128), lambda i: (0, i, 0)),
          pl.BlockSpec((NCORE, BR, 128), lambda i: (0, i, 0)),
          pl.BlockSpec((BR, 128), lambda i: (i, 0)),
          pl.BlockSpec((NCORE, NCORE, 128, 128), lambda i: (0, 0, 0, 0)),
          pl.BlockSpec((NCORE, 128), lambda i: (0, 0)),
      ],
      out_specs=[
          pl.BlockSpec((NCORE, BR, 128), lambda i: (0, i, 0)),
          pl.BlockSpec((NCORE, BR, 128), lambda i: (0, i, 0)),
      ],
      out_shape=[
          jax.ShapeDtypeStruct((NCORE, NP8, 128), jnp.float32),
          jax.ShapeDtypeStruct((NCORE, NP8, 128), jnp.float32),
      ],
  )(acc, selfpk, dishpk, kbd, b1t)


def _final_body(acc, selfpk, dishpk, b2t, wfc, bfc, out, scr):
  i = pl.program_id(0)
  dis = dishpk[...]
  rid = lax.broadcasted_iota(jnp.int32, (BR, 128), 0) + i * BR
  lid = lax.broadcasted_iota(jnp.int32, (BR, 128), 1) // HALF
  valid = rid * 8 + lid < N
  ninf = jnp.float32(-jnp.inf)
  for c in (0, 1):
    h = jnp.maximum(dis * acc[c] + selfpk[c] + b2t[c:c + 1], 0.0)
    h = jnp.where(valid, h, ninf)
    bmax = jnp.max(h, axis=0, keepdims=True)
    prev = jnp.where(i == 0, jnp.full((1, 128), ninf, jnp.float32),
                     scr[c:c + 1, :])
    scr[c:c + 1, :] = jnp.maximum(bmax, prev)

  @pl.when(i == GRID - 1)
  def _():
    pooled = []
    for c in (0, 1):
      p = scr[c:c + 1, :]
      m = p[:, 0:HALF]
      for k in range(1, 8):
        m = jnp.maximum(m, p[:, k * HALF:(k + 1) * HALF])
      pooled.append(m)
    pooled = jnp.concatenate(pooled, axis=1)
    logits = jnp.sum(pooled.reshape(HID, 1) * wfc[...], axis=0,
                     keepdims=True) + bfc[...]
    m = jnp.max(logits, axis=1, keepdims=True)
    z = logits - m
    out[...] = z - jnp.log(jnp.sum(jnp.exp(z), axis=1, keepdims=True))


def _tc_final(acc, selfpk, dishpk, b2t, wfc, bfc):
  return pl.pallas_call(
      _final_body,
      grid=(GRID,),
      in_specs=[
          pl.BlockSpec((NCORE, BR, 128), lambda i: (0, i, 0)),
          pl.BlockSpec((NCORE, BR, 128), lambda i: (0, i, 0)),
          pl.BlockSpec((BR, 128), lambda i: (i, 0)),
          pl.BlockSpec((NCORE, 128), lambda i: (0, 0)),
          pl.BlockSpec((HID, 5), lambda i: (0, 0)),
          pl.BlockSpec((1, 5), lambda i: (0, 0)),
      ],
      out_specs=pl.BlockSpec((1, 5), lambda i: (0, 0)),
      out_shape=jax.ShapeDtypeStruct((1, 5), jnp.float32),
      scratch_shapes=[pltpu.VMEM((8, 128), jnp.float32)],
  )(acc, selfpk, dishpk, b2t, wfc, bfc)


@jax.jit
def _run(x, edge_index, W1, b1, W2, b2, Wfc, bfc):
  src = edge_index[0].astype(jnp.int32)
  dst = edge_index[1].astype(jnp.int32)
  srcp1 = jnp.concatenate([src, _PAD_SRC]).reshape(ER, 128)
  dstp = jnp.concatenate([dst, _PAD_DST]).reshape(ER, 128)

  zeros = jnp.zeros((ZROWS, HALF), jnp.float32)
  ones = jnp.ones((128, HALF), jnp.float32)

  eye8 = jnp.eye(8, dtype=jnp.float32)
  w1bd = jnp.stack([jnp.kron(eye8, W1[:, :HALF]),
                    jnp.kron(eye8, W1[:, HALF:])])
  kbd = jnp.stack([
      jnp.stack([jnp.kron(eye8, W2[:HALF, :HALF]),
                 jnp.kron(eye8, W2[:HALF, HALF:])]),
      jnp.stack([jnp.kron(eye8, W2[HALF:, :HALF]),
                 jnp.kron(eye8, W2[HALF:, HALF:])]),
  ])
  b1t = jnp.stack([jnp.tile(b1[:HALF], 8), jnp.tile(b1[HALF:], 8)])
  b2t = jnp.stack([jnp.tile(b2[:HALF], 8), jnp.tile(b2[HALF:], 8)])

  x8 = x.reshape(N // 8, 64)

  ei3 = edge_index.astype(jnp.int32).reshape(2, DG_ROWS, 128)
  degh = _deg_call(ei3, zeros, ones)
  degh4 = degh.reshape(NCORE, NP8, 128)
  ypk, selfpk, dishpk = _tc_dense1(x8, degh4, w1bd)
  acc1 = _msg_call(ypk.reshape(2 * NP, HALF), srcp1, dstp, zeros)
  ypk2, selfpk2 = _tc_dense2(acc1.reshape(NCORE, NP8, 128), selfpk, dishpk,
                             kbd, b1t)
  acc2 = _msg_call(ypk2.reshape(2 * NP, HALF), srcp1, dstp, zeros)
  return _tc_final(acc2.reshape(NCORE, NP8, 128), selfpk2, dishpk, b2t,
                   Wfc, bfc.reshape(1, 5))


def kernel(x, edge_index, W1, b1, W2, b2, Wfc, bfc):
  return _run(x, edge_index, W1, b1, W2, b2, Wfc, bfc)


# speedup vs baseline: 1.0250x; 1.0250x over previous
import functools

import jax
import jax.numpy as jnp
from jax import lax
from jax.experimental import pallas as pl
from jax.experimental.pallas import tpu as pltpu
from jax.experimental.pallas import tpu_sc as plsc

N = 100000
E = 1600000
IN_F = 8
HID = 32
HALF = 16
NCORE = 2
NSUB = 16

NP = 100096
NV = NP
ZROWS = NV // NSUB
WROWS = NV // NSUB

EP = 1638400
PAD = EP - E
ER = EP // 128

import numpy as _np
_PAD_SRC = ((_np.arange(PAD) * 2621) % N).astype(_np.int32)
_PAD_DST = (N + (_np.arange(PAD) % 16)).astype(_np.int32)

MS_SLAB = 4
MS_RPS = ER // NSUB
MS_NSLAB = MS_RPS // MS_SLAB

DG_SLAB = 4
DG_RPS = ER // (NCORE * NSUB)
DG_NSLAB = DG_RPS // DG_SLAB

BN = 2944
GRID = NP // BN
BP = BN // 4
NPK = NP // 4
NP8 = NP // 8


_mesh = plsc.VectorSubcoreMesh(core_axis_name="c", subcore_axis_name="s")


def _deg_body(dstp, zeros, ones, degh, didx, ones_v, degS, sem_i, sem_s):
  c = lax.axis_index("c")
  s = lax.axis_index("s")
  pltpu.sync_copy(zeros, degS.at[pl.ds(s * ZROWS, ZROWS)])
  pltpu.sync_copy(ones, ones_v)
  row0 = (c * NSUB + s) * DG_RPS

  def idx_start(slab, q):
    pltpu.async_copy(
        dstp.at[pl.ds(row0 + slab * DG_SLAB, DG_SLAB)], didx.at[q],
        sem_i.at[q])

  def idx_wait(slab, q):
    pltpu.make_async_copy(
        dstp.at[pl.ds(row0 + slab * DG_SLAB, DG_SLAB)], didx.at[q],
        sem_i.at[q]).wait()

  def fire_scatters(b, q):
    for j in range(DG_SLAB):
      pltpu.async_copy(ones_v, degS.at[didx.at[q, j]], sem_s.at[b],
                       add=True)

  def drain_scatters(b, q):
    for j in range(DG_SLAB):
      pltpu.make_async_copy(ones_v, degS.at[didx.at[q, j]],
                            sem_s.at[b]).wait()

  idx_start(0, 0)
  idx_start(1, 1)
  plsc.subcore_barrier()

  def outer(g, carry):
    for k in range(4):
      i = g * 4 + k
      b = k % 2
      q = k % 4
      idx_wait(i, q)
      if k < 2:
        @pl.when(g > 0)
        def _():
          drain_scatters(b, (k - 2) % 4)
      else:
        drain_scatters(b, (k - 2) % 4)
      fire_scatters(b, q)

      @pl.when(i + 2 < DG_NSLAB)
      def _():
        idx_start(i + 2, (k + 2) % 4)
    return carry

  lax.fori_loop(0, DG_NSLAB // 4, outer, 0)
  drain_scatters(0, 2)
  drain_scatters(1, 3)
  plsc.subcore_barrier()
  pltpu.sync_copy(degS.at[pl.ds(s * WROWS, WROWS)],
                  degh.at[c, pl.ds(s * WROWS, WROWS)])


_deg_call = pl.kernel(
    _deg_body,
    out_type=jax.ShapeDtypeStruct((NCORE, NP, HALF), jnp.float32),
    mesh=_mesh,
    compiler_params=pltpu.CompilerParams(use_tc_tiling_on_sc=False),
    scratch_types=[
        pltpu.VMEM((4, DG_SLAB, 128), jnp.int32),
        pltpu.VMEM((128, HALF), jnp.float32),
        pltpu.VMEM_SHARED((NV, HALF), jnp.float32),
        pltpu.SemaphoreType.DMA((4,)),
        pltpu.SemaphoreType.DMA((2,)),
    ],
)


def _msg_body(ycat, srcp1, dstp, zeros, acc, sidx, didx, rows, accS,
              sem_g, sem_i, sem_s):
  c = lax.axis_index("c")
  s = lax.axis_index("s")
  pltpu.sync_copy(zeros, accS.at[pl.ds(s * ZROWS, ZROWS)])
  row0 = s * MS_RPS

  off = c * NP

  def idx_start(slab, q):
    r0 = row0 + slab * MS_SLAB
    pltpu.async_copy(srcp1.at[pl.ds(r0, MS_SLAB)], sidx.at[q],
                     sem_i.at[q])
    pltpu.async_copy(dstp.at[pl.ds(r0, MS_SLAB)], didx.at[q], sem_i.at[q])

  def idx_wait(slab, q):
    r0 = row0 + slab * MS_SLAB
    pltpu.make_async_copy(srcp1.at[pl.ds(r0, MS_SLAB)], sidx.at[q],
                          sem_i.at[q]).wait()
    pltpu.make_async_copy(dstp.at[pl.ds(r0, MS_SLAB)], didx.at[q],
                          sem_i.at[q]).wait()
    for r in range(MS_SLAB):
      for t in range(8):
        sl = pl.ds(t * 16, 16)
        sidx[q, r, sl] = sidx[q, r, sl] + off

  def fire_gathers(b, q):
    for j in range(MS_SLAB):
      pltpu.async_copy(ycat.at[sidx.at[q, j]],
                       rows.at[b, pl.ds(j * 128, 128)], sem_g.at[b])

  def drain_gathers(b, q):
    for j in range(MS_SLAB):
      pltpu.make_async_copy(ycat.at[sidx.at[q, j]],
                            rows.at[b, pl.ds(j * 128, 128)],
                            sem_g.at[b]).wait()

  def fire_scatters(b, q):
    for j in range(MS_SLAB):
      pltpu.async_copy(rows.at[b, pl.ds(j * 128, 128)],
                       accS.at[didx.at[q, j]], sem_s.at[b], add=True)

  def drain_scatters(b, q):
    for j in range(MS_SLAB):
      pltpu.make_async_copy(rows.at[b, pl.ds(j * 128, 128)],
                            accS.at[didx.at[q, j]], sem_s.at[b]).wait()

  idx_start(0, 0)
  idx_wait(0, 0)
  fire_gathers(0, 0)
  idx_start(1, 1)
  plsc.subcore_barrier()

  def outer(g, carry):
    for k in range(4):
      i = g * 4 + k
      b = k % 2
      q = k % 4
      drain_gathers(b, q)
      if k == 0:
        @pl.when(g > 0)
        def _():
          drain_scatters(1 - b, (k - 1) % 4)
      else:
        drain_scatters(1 - b, (k - 1) % 4)
      if k == 3:
        @pl.when(g < MS_NSLAB // 4 - 1)
        def _():
          idx_wait(i + 1, (k + 1) % 4)
          fire_gathers(1 - b, (k + 1) % 4)
      else:
        idx_wait(i + 1, (k + 1) % 4)
        fire_gathers(1 - b, (k + 1) % 4)
      fire_scatters(b, q)

      @pl.when(i + 2 < MS_NSLAB)
      def _():
        idx_start(i + 2, (k + 2) % 4)
    return carry

  lax.fori_loop(0, MS_NSLAB // 4, outer, 0)
  drain_scatters(1, 3)
  plsc.subcore_barrier()
  pltpu.sync_copy(accS.at[pl.ds(s * WROWS, WROWS)],
                  acc.at[c, pl.ds(s * WROWS, WROWS)])


_msg_call = pl.kernel(
    _msg_body,
    out_type=jax.ShapeDtypeStruct((NCORE, NP, HALF), jnp.float32),
    mesh=_mesh,
    compiler_params=pltpu.CompilerParams(use_tc_tiling_on_sc=False),
    scratch_types=[
        pltpu.VMEM((4, MS_SLAB, 128), jnp.int32),
        pltpu.VMEM((4, MS_SLAB, 128), jnp.int32),
        pltpu.VMEM((2, MS_SLAB * 128, HALF), jnp.float32),
        pltpu.VMEM_SHARED((NV, HALF), jnp.float32),
        pltpu.SemaphoreType.DMA((2,)),
        pltpu.SemaphoreType.DMA((4,)),
        pltpu.SemaphoreType.DMA((2,)),
    ],
)


def _dense1_body(x8, degh4, w1bd, ypk, selfpk, dishpk):
  dis = lax.rsqrt(degh4[0] + degh4[1] + 1.0)
  y0 = jnp.dot(x8[...], w1bd[0], preferred_element_type=jnp.float32) * dis
  y1 = jnp.dot(x8[...], w1bd[1], preferred_element_type=jnp.float32) * dis
  ypk[...] = jnp.concatenate([y0[None], y1[None]], axis=0)
  selfpk[...] = jnp.concatenate([(y0 * dis)[None], (y1 * dis)[None]], axis=0)
  dishpk[...] = dis


BR = NP8 // GRID


def _tc_dense1(x8, degh4, w1bd):
  return pl.pallas_call(
      _dense1_body,
      grid=(GRID,),
      in_specs=[
          pl.BlockSpec((BR, 64), lambda i: (i, 0)),
          pl.BlockSpec((NCORE, BR, 128), lambda i: (0, i, 0)),
          pl.BlockSpec((NCORE, 64, 128), lambda i: (0, 0, 0)),
      ],
      out_specs=[
          pl.BlockSpec((NCORE, BR, 128), lambda i: (0, i, 0)),
          pl.BlockSpec((NCORE, BR, 128), lambda i: (0, i, 0)),
          pl.BlockSpec((BR, 128), lambda i: (i, 0)),
      ],
      out_shape=[
          jax.ShapeDtypeStruct((NCORE, NP8, 128), jnp.float32),
          jax.ShapeDtypeStruct((NCORE, NP8, 128), jnp.float32),
          jax.ShapeDtypeStruct((NP8, 128), jnp.float32),
      ],
  )(x8, degh4, w1bd)


def _dense2_body(acc, selfpk, dishpk, kbd, b1t, ypk2, selfpk2):
  dis = dishpk[...]
  h0 = jnp.maximum(dis * acc[0] + selfpk[0] + b1t[0:1], 0.0)
  h1 = jnp.maximum(dis * acc[1] + selfpk[1] + b1t[1:2], 0.0)
  y20 = (jnp.dot(h0, kbd[0, 0], preferred_element_type=jnp.float32)
         + jnp.dot(h1, kbd[1, 0], preferred_element_type=jnp.float32)) * dis
  y21 = (jnp.dot(h0, kbd[0, 1], preferred_element_type=jnp.float32)
         + jnp.dot(h1, kbd[1, 1], preferred_element_type=jnp.float32)) * dis
  ypk2[...] = jnp.concatenate([y20[None], y21[None]], axis=0)
  selfpk2[...] = jnp.concatenate([(y20 * dis)[None], (y21 * dis)[None]],
                                 axis=0)


def _tc_dense2(acc, selfpk, dishpk, kbd, b1t):
  return pl.pallas_call(
      _dense2_body,
      grid=(GRID,),
      in_specs=[
          pl.BlockSpec((NCORE, BR, 128), lambda i: (0, i, 0)),
          pl.BlockSpec((NCORE, BR, 128), lambda i: (0, i, 0)),
          pl.BlockSpec((BR, 128), lambda i: (i, 0)),
          pl.BlockSpec((NCORE, NCORE, 128, 128), lambda i: (0, 0, 0, 0)),
          pl.BlockSpec((NCORE, 128), lambda i: (0, 0)),
      ],
      out_specs=[
          pl.BlockSpec((NCORE, BR, 128), lambda i: (0, i, 0)),
          pl.BlockSpec((NCORE, BR, 128), lambda i: (0, i, 0)),
      ],
      out_shape=[
          jax.ShapeDtypeStruct((NCORE, NP8, 128), jnp.float32),
          jax.ShapeDtypeStruct((NCORE, NP8, 128), jnp.float32),
      ],
  )(acc, selfpk, dishpk, kbd, b1t)


def _final_body(acc, selfpk, dishpk, b2t, wfc, bfc, out, scr):
  i = pl.program_id(0)
  dis = dishpk[...]
  rid = lax.broadcasted_iota(jnp.int32, (BR, 128), 0) + i * BR
  lid = lax.broadcasted_iota(jnp.int32, (BR, 128), 1) // HALF
  valid = rid * 8 + lid < N
  ninf = jnp.float32(-jnp.inf)
  for c in (0, 1):
    h = jnp.maximum(dis * acc[c] + selfpk[c] + b2t[c:c + 1], 0.0)
    h = jnp.where(valid, h, ninf)
    bmax = jnp.max(h, axis=0, keepdims=True)
    prev = jnp.where(i == 0, jnp.full((1, 128), ninf, jnp.float32),
                     scr[c:c + 1, :])
    scr[c:c + 1, :] = jnp.maximum(bmax, prev)

  @pl.when(i == GRID - 1)
  def _():
    pooled = []
    for c in (0, 1):
      p = scr[c:c + 1, :]
      m = p[:, 0:HALF]
      for k in range(1, 8):
        m = jnp.maximum(m, p[:, k * HALF:(k + 1) * HALF])
      pooled.append(m)
    pooled = jnp.concatenate(pooled, axis=1)
    logits = jnp.sum(pooled.reshape(HID, 1) * wfc[...], axis=0,
                     keepdims=True) + bfc[...]
    m = jnp.max(logits, axis=1, keepdims=True)
    z = logits - m
    out[...] = z - jnp.log(jnp.sum(jnp.exp(z), axis=1, keepdims=True))


def _tc_final(acc, selfpk, dishpk, b2t, wfc, bfc):
  return pl.pallas_call(
      _final_body,
      grid=(GRID,),
      in_specs=[
          pl.BlockSpec((NCORE, BR, 128), lambda i: (0, i, 0)),
          pl.BlockSpec((NCORE, BR, 128), lambda i: (0, i, 0)),
          pl.BlockSpec((BR, 128), lambda i: (i, 0)),
          pl.BlockSpec((NCORE, 128), lambda i: (0, 0)),
          pl.BlockSpec((HID, 5), lambda i: (0, 0)),
          pl.BlockSpec((1, 5), lambda i: (0, 0)),
      ],
      out_specs=pl.BlockSpec((1, 5), lambda i: (0, 0)),
      out_shape=jax.ShapeDtypeStruct((1, 5), jnp.float32),
      scratch_shapes=[pltpu.VMEM((8, 128), jnp.float32)],
  )(acc, selfpk, dishpk, b2t, wfc, bfc)


@jax.jit
def _run(x, edge_index, W1, b1, W2, b2, Wfc, bfc):
  src = edge_index[0].astype(jnp.int32)
  dst = edge_index[1].astype(jnp.int32)
  srcp1 = jnp.concatenate([src, _PAD_SRC]).reshape(ER, 128)
  dstp = jnp.concatenate([dst, _PAD_DST]).reshape(ER, 128)

  zeros = jnp.zeros((ZROWS, HALF), jnp.float32)
  ones = jnp.ones((128, HALF), jnp.float32)

  eye8 = jnp.eye(8, dtype=jnp.float32)
  w1bd = jnp.stack([jnp.kron(eye8, W1[:, :HALF]),
                    jnp.kron(eye8, W1[:, HALF:])])
  kbd = jnp.stack([
      jnp.stack([jnp.kron(eye8, W2[:HALF, :HALF]),
                 jnp.kron(eye8, W2[:HALF, HALF:])]),
      jnp.stack([jnp.kron(eye8, W2[HALF:, :HALF]),
                 jnp.kron(eye8, W2[HALF:, HALF:])]),
  ])
  b1t = jnp.stack([jnp.tile(b1[:HALF], 8), jnp.tile(b1[HALF:], 8)])
  b2t = jnp.stack([jnp.tile(b2[:HALF], 8), jnp.tile(b2[HALF:], 8)])

  x8 = x.reshape(N // 8, 64)

  degh = _deg_call(dstp, zeros, ones)
  degh4 = degh.reshape(NCORE, NP8, 128)
  ypk, selfpk, dishpk = _tc_dense1(x8, degh4, w1bd)
  acc1 = _msg_call(ypk.reshape(2 * NP, HALF), srcp1, dstp, zeros)
  ypk2, selfpk2 = _tc_dense2(acc1.reshape(NCORE, NP8, 128), selfpk, dishpk,
                             kbd, b1t)
  acc2 = _msg_call(ypk2.reshape(2 * NP, HALF), srcp1, dstp, zeros)
  return _tc_final(acc2.reshape(NCORE, NP8, 128), selfpk2, dishpk, b2t,
                   Wfc, bfc.reshape(1, 5))


def kernel(x, edge_index, W1, b1, W2, b2, Wfc, bfc):
  return _run(x, edge_index, W1, b1, W2, b2, Wfc, bfc)
